# Initial kernel scaffold; baseline (speedup 1.0000x reference)
import jax
import jax.numpy as jnp
from jax.experimental import pallas as pl


def kernel(x, edges):
    raise NotImplementedError("write your pallas kernel here")


# speedup vs baseline: 1.6418x; 1.6418x over previous
import functools

import jax
import jax.numpy as jnp
from jax import lax
from jax.experimental import pallas as pl
from jax.experimental.pallas import tpu as pltpu


def _routing_chunk_kernel(cs_ref, ce_ref,
                          z_ref,
                          zt_hbm, src_hbm,
                          c_ref,
                          zt0, src0, zt0f, oh_ref,
                          ztw, srcw, dma_sem,
                          z_loc, c_loc, agg,
                          *, routit, inv_tau, tn, w, k, d, eps):
    kd = k * d
    i = pl.program_id(0)
    base = i * tn
    e_lo = cs_ref[i]
    e_hi = ce_ref[i]
    nw = (e_hi - e_lo + w - 1) // w

    e0 = pl.multiple_of(e_lo, 128)
    cp_z = pltpu.make_async_copy(zt_hbm.at[:, pl.ds(e0, w)], zt0, dma_sem.at[0])
    cp_s = pltpu.make_async_copy(src_hbm.at[:, pl.ds(e0, w)], src0, dma_sem.at[1])
    cp_z.start()
    cp_s.start()

    zl = z_ref[...].astype(jnp.float32).T
    z_loc[...] = zl
    c_loc[...] = zl
    ids = lax.broadcasted_iota(jnp.int32, (tn, w), 0) + base

    cp_z.wait()
    cp_s.wait()
    zt0f[...] = zt0[...].astype(jnp.float32)
    oh_ref[...] = (ids == src0[...]).astype(jnp.bfloat16)

    def one_routing_iter(_, carry):
        c_bf = c_loc[...].astype(jnp.bfloat16)

        zf = zt0f[...]
        c_src = jnp.dot(c_bf, oh_ref[...],
                        preferred_element_type=jnp.float32)
        logits = jnp.sum((c_src * zf).reshape(k, d, w), axis=1,
                         keepdims=True) * inv_tau
        mx = jnp.max(logits, axis=0, keepdims=True)
        ex = jnp.exp(logits - mx)
        p = ex / jnp.sum(ex, axis=0, keepdims=True)
        ws = (p * zf.reshape(k, d, w)).reshape(kd, w).astype(jnp.bfloat16)
        agg[...] = lax.dot_general(ws, oh_ref[...], (((1,), (1,)), ((), ())),
                                   preferred_element_type=jnp.float32)

        def wbody(wi, carry2):
            ew = pl.multiple_of(e_lo + wi * w, 128)
            cp_z2 = pltpu.make_async_copy(zt_hbm.at[:, pl.ds(ew, w)], ztw,
                                          dma_sem.at[0])
            cp_s2 = pltpu.make_async_copy(src_hbm.at[:, pl.ds(ew, w)], srcw,
                                          dma_sem.at[1])
            cp_z2.start()
            cp_s2.start()
            cp_z2.wait()
            cp_s2.wait()
            ohw = (ids == srcw[...]).astype(jnp.bfloat16)
            zfw = ztw[...].astype(jnp.float32)
            c_src2 = jnp.dot(c_bf, ohw, preferred_element_type=jnp.float32)
            lg2 = jnp.sum((c_src2 * zfw).reshape(k, d, w), axis=1,
                          keepdims=True) * inv_tau
            mx2 = jnp.max(lg2, axis=0, keepdims=True)
            ex2 = jnp.exp(lg2 - mx2)
            p2 = ex2 / jnp.sum(ex2, axis=0, keepdims=True)
            ws2 = (p2 * zfw.reshape(k, d, w)).reshape(kd, w).astype(jnp.bfloat16)
            agg[...] += lax.dot_general(ws2, ohw, (((1,), (1,)), ((), ())),
                                        preferred_element_type=jnp.float32)
            return carry2

        lax.fori_loop(1, nw, wbody, 0)

        cn = (z_loc[...] + agg[...]).reshape(k, d, tn)
        ss = jnp.sum(cn * cn, axis=1, keepdims=True)
        inv = lax.rsqrt(jnp.maximum(ss, eps * eps))
        c_loc[...] = (cn * inv).reshape(kd, tn)
        return carry

    lax.fori_loop(0, routit, one_routing_iter, 0)
    c_ref[...] = c_loc[...].T


def _round_up(x, mult):
    return ((x + mult - 1) // mult) * mult


def _routing_layer(x, edges, *, num_factors, routit, tau):
    n, k, d = x.shape
    assert k == num_factors
    kd = k * d
    src = edges[0].astype(jnp.int32)
    trg = edges[1].astype(jnp.int32)
    m = int(src.shape[0])

    tn = 256
    w = 3072
    n_pad = _round_up(n, tn)
    num_chunks = n_pad // tn
    m_pad = _round_up(max(m, 1), w) + w

    order = jnp.argsort(src)
    src_s = jnp.take(src, order)
    trg_s = jnp.take(trg, order)
    rowptr = jnp.searchsorted(src_s, jnp.arange(n + 1, dtype=jnp.int32)).astype(jnp.int32)

    chunk_lo = jnp.minimum(jnp.arange(num_chunks, dtype=jnp.int32) * tn, n)
    chunk_hi = jnp.minimum(chunk_lo + tn, n)
    chunk_e_start = (rowptr[chunk_lo] // 128) * 128
    chunk_e_end = rowptr[chunk_hi]

    z_rows = x.reshape(n, kd)
    if n_pad > n:
        z_in = jnp.zeros((n_pad, kd), x.dtype).at[:n].set(z_rows)
    else:
        z_in = z_rows
    zt_rows = jnp.take(z_rows.astype(jnp.bfloat16), trg_s, axis=0)
    zt_e = jnp.zeros((kd, m_pad), jnp.bfloat16).at[:, :m].set(zt_rows.T)
    src_pad = jnp.full((1, m_pad), -1, dtype=jnp.int32).at[0, :m].set(src_s)

    _kernel_fn = functools.partial(_routing_chunk_kernel, routit=int(routit),
                                   inv_tau=float(1.0 / tau), tn=tn, w=w,
                                   k=k, d=d, eps=1e-12)

    c_out = pl.pallas_call(
        _kernel_fn,
        out_shape=jax.ShapeDtypeStruct((n_pad, kd), jnp.float32),
        grid_spec=pltpu.PrefetchScalarGridSpec(
            num_scalar_prefetch=2,
            grid=(num_chunks,),
            in_specs=[pl.BlockSpec((tn, kd), lambda i, cs, ce: (i, 0)),
                      pl.BlockSpec(memory_space=pl.ANY),
                      pl.BlockSpec(memory_space=pl.ANY)],
            out_specs=pl.BlockSpec((tn, kd), lambda i, cs, ce: (i, 0)),
            scratch_shapes=[pltpu.VMEM((kd, w), jnp.bfloat16),
                            pltpu.VMEM((1, w), jnp.int32),
                            pltpu.VMEM((kd, w), jnp.float32),
                            pltpu.VMEM((tn, w), jnp.bfloat16),
                            pltpu.VMEM((kd, w), jnp.bfloat16),
                            pltpu.VMEM((1, w), jnp.int32),
                            pltpu.SemaphoreType.DMA((2,)),
                            pltpu.VMEM((kd, tn), jnp.float32),
                            pltpu.VMEM((kd, tn), jnp.float32),
                            pltpu.VMEM((kd, tn), jnp.float32)]),
        compiler_params=pltpu.CompilerParams(
            dimension_semantics=("parallel",),
            vmem_limit_bytes=32 * 1024 * 1024),
    )(chunk_e_start, chunk_e_end, z_in, zt_e, src_pad)

    return c_out[:n].reshape(n, k, d)


def kernel(x, edges):
    return _routing_layer(x, edges, num_factors=8, routit=6, tau=1.0)


# speedup vs baseline: 3.6097x; 2.1986x over previous
import functools

import jax
import jax.numpy as jnp
from jax import lax
from jax.experimental import pallas as pl
from jax.experimental.pallas import tpu as pltpu


def _routing_chunk_kernel(cs_ref, ce_ref,
                          z_ref,
                          zt_hbm, src_hbm,
                          c_ref,
                          zt0, src0, zt0f, oh_ref,
                          ztw, srcw, dma_sem,
                          z_loc, c_loc, agg,
                          *, routit, inv_tau, tn, w, k, d, eps):
    kd = k * d
    i = pl.program_id(0)
    base = i * tn
    e_lo = cs_ref[i]
    e_hi = ce_ref[i]
    nw = (e_hi - e_lo + w - 1) // w

    e0 = pl.multiple_of(e_lo, 128)
    cp_z = pltpu.make_async_copy(zt_hbm.at[:, pl.ds(e0, w)], zt0, dma_sem.at[0])
    cp_s = pltpu.make_async_copy(src_hbm.at[:, pl.ds(e0, w)], src0, dma_sem.at[1])
    cp_z.start()
    cp_s.start()

    zl = z_ref[...].astype(jnp.float32).T
    z_loc[...] = zl
    c_loc[...] = zl
    ids = lax.broadcasted_iota(jnp.int32, (tn, w), 0) + base

    cp_z.wait()
    cp_s.wait()
    zt0f[...] = zt0[...].astype(jnp.float32)
    oh_ref[...] = (ids == src0[...]).astype(jnp.bfloat16)

    def one_routing_iter(_, carry):
        c_bf = c_loc[...].astype(jnp.bfloat16)

        zf = zt0f[...]
        c_src = jnp.dot(c_bf, oh_ref[...],
                        preferred_element_type=jnp.float32)
        logits = jnp.sum((c_src * zf).reshape(k, d, w), axis=1,
                         keepdims=True) * inv_tau
        mx = jnp.max(logits, axis=0, keepdims=True)
        ex = jnp.exp(logits - mx)
        p = ex / jnp.sum(ex, axis=0, keepdims=True)
        ws = (p * zf.reshape(k, d, w)).reshape(kd, w).astype(jnp.bfloat16)
        agg[...] = lax.dot_general(ws, oh_ref[...], (((1,), (1,)), ((), ())),
                                   preferred_element_type=jnp.float32)

        def wbody(wi, carry2):
            ew = pl.multiple_of(e_lo + wi * w, 128)
            cp_z2 = pltpu.make_async_copy(zt_hbm.at[:, pl.ds(ew, w)], ztw,
                                          dma_sem.at[0])
            cp_s2 = pltpu.make_async_copy(src_hbm.at[:, pl.ds(ew, w)], srcw,
                                          dma_sem.at[1])
            cp_z2.start()
            cp_s2.start()
            cp_z2.wait()
            cp_s2.wait()
            ohw = (ids == srcw[...]).astype(jnp.bfloat16)
            zfw = ztw[...].astype(jnp.float32)
            c_src2 = jnp.dot(c_bf, ohw, preferred_element_type=jnp.float32)
            lg2 = jnp.sum((c_src2 * zfw).reshape(k, d, w), axis=1,
                          keepdims=True) * inv_tau
            mx2 = jnp.max(lg2, axis=0, keepdims=True)
            ex2 = jnp.exp(lg2 - mx2)
            p2 = ex2 / jnp.sum(ex2, axis=0, keepdims=True)
            ws2 = (p2 * zfw.reshape(k, d, w)).reshape(kd, w).astype(jnp.bfloat16)
            agg[...] += lax.dot_general(ws2, ohw, (((1,), (1,)), ((), ())),
                                        preferred_element_type=jnp.float32)
            return carry2

        lax.fori_loop(1, nw, wbody, 0)

        cn = (z_loc[...] + agg[...]).reshape(k, d, tn)
        ss = jnp.sum(cn * cn, axis=1, keepdims=True)
        inv = lax.rsqrt(jnp.maximum(ss, eps * eps))
        c_loc[...] = (cn * inv).reshape(kd, tn)
        return carry

    lax.fori_loop(0, routit, one_routing_iter, 0)
    c_ref[...] = c_loc[...].T


def _round_up(x, mult):
    return ((x + mult - 1) // mult) * mult


def _routing_layer(x, edges, *, num_factors, routit, tau):
    n, k, d = x.shape
    assert k == num_factors
    kd = k * d
    src = edges[0].astype(jnp.int32)
    trg = edges[1].astype(jnp.int32)
    m = int(src.shape[0])

    tn = 256
    w = 3072
    n_pad = _round_up(n, tn)
    num_chunks = n_pad // tn
    m_pad = _round_up(max(m, 1), w) + w

    order = jnp.argsort(src)
    src_s = jnp.take(src, order)
    trg_s = jnp.take(trg, order)
    bounds = jnp.minimum(jnp.arange(num_chunks + 1, dtype=jnp.int32) * tn, n)
    ptr = jnp.searchsorted(src_s, bounds).astype(jnp.int32)
    chunk_e_start = (ptr[:-1] // 128) * 128
    chunk_e_end = ptr[1:]

    z_rows = x.reshape(n, kd)
    if n_pad > n:
        z_in = jnp.zeros((n_pad, kd), x.dtype).at[:n].set(z_rows)
    else:
        z_in = z_rows
    zt_rows = jnp.take(z_rows.astype(jnp.bfloat16), trg_s, axis=0)
    zt_e = jnp.zeros((kd, m_pad), jnp.bfloat16).at[:, :m].set(zt_rows.T)
    src_pad = jnp.full((1, m_pad), -1, dtype=jnp.int32).at[0, :m].set(src_s)

    _kernel_fn = functools.partial(_routing_chunk_kernel, routit=int(routit),
                                   inv_tau=float(1.0 / tau), tn=tn, w=w,
                                   k=k, d=d, eps=1e-12)

    c_out = pl.pallas_call(
        _kernel_fn,
        out_shape=jax.ShapeDtypeStruct((n_pad, kd), jnp.float32),
        grid_spec=pltpu.PrefetchScalarGridSpec(
            num_scalar_prefetch=2,
            grid=(num_chunks,),
            in_specs=[pl.BlockSpec((tn, kd), lambda i, cs, ce: (i, 0)),
                      pl.BlockSpec(memory_space=pl.ANY),
                      pl.BlockSpec(memory_space=pl.ANY)],
            out_specs=pl.BlockSpec((tn, kd), lambda i, cs, ce: (i, 0)),
            scratch_shapes=[pltpu.VMEM((kd, w), jnp.bfloat16),
                            pltpu.VMEM((1, w), jnp.int32),
                            pltpu.VMEM((kd, w), jnp.float32),
                            pltpu.VMEM((tn, w), jnp.bfloat16),
                            pltpu.VMEM((kd, w), jnp.bfloat16),
                            pltpu.VMEM((1, w), jnp.int32),
                            pltpu.SemaphoreType.DMA((2,)),
                            pltpu.VMEM((kd, tn), jnp.float32),
                            pltpu.VMEM((kd, tn), jnp.float32),
                            pltpu.VMEM((kd, tn), jnp.float32)]),
        compiler_params=pltpu.CompilerParams(
            dimension_semantics=("parallel",),
            vmem_limit_bytes=32 * 1024 * 1024),
    )(chunk_e_start, chunk_e_end, z_in, zt_e, src_pad)

    return c_out[:n].reshape(n, k, d)


def kernel(x, edges):
    return _routing_layer(x, edges, num_factors=8, routit=6, tau=1.0)
